# baseline (speedup 1.0000x reference)
import functools

import jax
import jax.numpy as jnp
from jax import lax
from jax.experimental import pallas as pl
from jax.experimental.pallas import tpu as pltpu
from jax.experimental.pallas import tpu_sc as plsc

B = 16384
D = 64
NC = 2
NS = 16
NW = NC * NS
BPW = B // NW
CHUNK = 128
NCHUNK = BPW // CHUNK
GROUPS = CHUNK // 16


def _mf_body(item_idx_hbm, user_idx_hbm, item_tab, user_tab, out_hbm,
             ii_v, ui_v, iv_rows, uv_rows, out_v, sem_i, sem_u):
    wid = lax.axis_index("s") * NC + lax.axis_index("c")
    base = wid * BPW

    pltpu.sync_copy(item_idx_hbm.at[wid], ii_v)
    pltpu.sync_copy(user_idx_hbm.at[wid], ui_v)

    iota16 = lax.iota(jnp.int32, 16)
    perm = {sh: iota16 ^ sh for sh in (8, 4, 2, 1)}

    for k in range(NCHUNK):
        cp_i = pltpu.async_copy(item_tab.at[ii_v.at[k]], iv_rows, sem_i)
        cp_u = pltpu.async_copy(user_tab.at[ui_v.at[k]], uv_rows, sem_u)
        cp_i.wait()
        cp_u.wait()

        def group_body(g, carry, k=k):
            rb = g * 16
            acc = jnp.zeros((16,), jnp.float32)
            for j in range(16):
                r = rb + j
                p = iv_rows[r, pl.ds(0, 16)] * uv_rows[r, pl.ds(0, 16)]
                p = p + iv_rows[r, pl.ds(16, 16)] * uv_rows[r, pl.ds(16, 16)]
                p = p + iv_rows[r, pl.ds(32, 16)] * uv_rows[r, pl.ds(32, 16)]
                p = p + iv_rows[r, pl.ds(48, 16)] * uv_rows[r, pl.ds(48, 16)]
                for sh in (8, 4, 2, 1):
                    p = p + p.at[perm[sh]].get(mode="promise_in_bounds")
                acc = jnp.where(iota16 == j, p, acc)
            out_v[pl.ds(k * CHUNK + rb, 16)] = acc
            return carry

        lax.fori_loop(0, GROUPS, group_body, 0)

    pltpu.sync_copy(out_v, out_hbm.at[pl.ds(base, BPW)])


@functools.partial(jax.jit, static_argnames=())
def _mf(item_idx, user_idx, item_table, user_table):
    mesh = plsc.VectorSubcoreMesh(core_axis_name="c", subcore_axis_name="s")
    kern = functools.partial(
        pl.kernel,
        mesh=mesh,
        compiler_params=pltpu.CompilerParams(use_tc_tiling_on_sc=False),
        out_type=jax.ShapeDtypeStruct((B,), jnp.float32),
        scratch_types=[
            pltpu.VMEM((NCHUNK, CHUNK), jnp.int32),
            pltpu.VMEM((NCHUNK, CHUNK), jnp.int32),
            pltpu.VMEM((CHUNK, D), jnp.float32),
            pltpu.VMEM((CHUNK, D), jnp.float32),
            pltpu.VMEM((BPW,), jnp.float32),
            pltpu.SemaphoreType.DMA,
            pltpu.SemaphoreType.DMA,
        ],
    )(_mf_body)
    return kern(item_idx, user_idx, item_table, user_table)


def kernel(item_vec, user_vec, item_table, user_table):
    item_idx = item_vec.reshape(NW, NCHUNK, CHUNK)
    user_idx = user_vec.reshape(NW, NCHUNK, CHUNK)
    return _mf(item_idx, user_idx, item_table, user_table)
